# baseline (speedup 1.0000x reference)
import jax
import jax.numpy as jnp
from jax.experimental import pallas as pl
from jax.experimental.pallas import tpu as pltpu

BATCH = 64
HEIGHT = 32
WIDTH = 32
EMBED_DIM = 256
MAX_POS = 50

BLOCK_B = 2


def _body(row_ref, col_ref, out_ref):
    r = pl.program_id(1)
    x = col_ref[:WIDTH, :]
    y = row_ref[pl.ds(r * (HEIGHT // 2), HEIGHT // 2), :]
    left = jnp.broadcast_to(x[None, :, :], (HEIGHT // 2, WIDTH, EMBED_DIM))
    left = left.reshape(HEIGHT * WIDTH // 2, EMBED_DIM)
    right = jnp.broadcast_to(y[:, None, :], (HEIGHT // 2, WIDTH, EMBED_DIM))
    right = right.reshape(HEIGHT * WIDTH // 2, EMBED_DIM)
    tile = jnp.concatenate([left, right], axis=-1)
    out_ref[...] = jnp.broadcast_to(tile[None], (BLOCK_B, HEIGHT * WIDTH // 2, 2 * EMBED_DIM))


def kernel(row_embeddings, column_embeddings):
    out = pl.pallas_call(
        _body,
        grid=(BATCH // BLOCK_B, 2),
        in_specs=[
            pl.BlockSpec((MAX_POS, EMBED_DIM), lambda b, r: (0, 0)),
            pl.BlockSpec((MAX_POS, EMBED_DIM), lambda b, r: (0, 0)),
        ],
        out_specs=pl.BlockSpec(
            (BLOCK_B, HEIGHT * WIDTH // 2, 2 * EMBED_DIM), lambda b, r: (b, r, 0)
        ),
        out_shape=jax.ShapeDtypeStruct(
            (BATCH, HEIGHT * WIDTH, 2 * EMBED_DIM), jnp.float32
        ),
        compiler_params=pltpu.CompilerParams(
            dimension_semantics=("arbitrary", "arbitrary"),
        ),
    )(row_embeddings, column_embeddings)
    return out


# speedup vs baseline: 1.1409x; 1.1409x over previous
import jax
import jax.numpy as jnp
from jax.experimental import pallas as pl
from jax.experimental.pallas import tpu as pltpu

BATCH = 64
HEIGHT = 32
WIDTH = 32
EMBED_DIM = 256
MAX_POS = 50

REP = 4
NSEM = 4


def _body(row_ref, col_ref, out_hbm, tile_v, sems):
    x = col_ref[:WIDTH, :]
    y = row_ref[:HEIGHT, :]
    left = jnp.broadcast_to(x[None, :, :], (HEIGHT, WIDTH, EMBED_DIM))
    left = left.reshape(HEIGHT * WIDTH, EMBED_DIM)
    right = jnp.broadcast_to(y[:, None, :], (HEIGHT, WIDTH, EMBED_DIM))
    right = right.reshape(HEIGHT * WIDTH, EMBED_DIM)
    tile = jnp.concatenate([left, right], axis=-1)
    tile_v[...] = jnp.broadcast_to(
        tile[None], (REP, HEIGHT * WIDTH, 2 * EMBED_DIM)
    )
    copies = [
        pltpu.make_async_copy(
            tile_v, out_hbm.at[pl.ds(b * REP, REP)], sems.at[b % NSEM]
        )
        for b in range(BATCH // REP)
    ]
    for cp in copies:
        cp.start()
    for cp in copies:
        cp.wait()


def kernel(row_embeddings, column_embeddings):
    out = pl.pallas_call(
        _body,
        in_specs=[
            pl.BlockSpec(memory_space=pltpu.VMEM),
            pl.BlockSpec(memory_space=pltpu.VMEM),
        ],
        out_specs=pl.BlockSpec(memory_space=pl.ANY),
        out_shape=jax.ShapeDtypeStruct(
            (BATCH, HEIGHT * WIDTH, 2 * EMBED_DIM), jnp.float32
        ),
        scratch_shapes=[
            pltpu.VMEM((REP, HEIGHT * WIDTH, 2 * EMBED_DIM), jnp.float32),
            pltpu.SemaphoreType.DMA((NSEM,)),
        ],
    )(row_embeddings, column_embeddings)
    return out


# speedup vs baseline: 1.1976x; 1.0498x over previous
import jax
import jax.numpy as jnp
from jax.experimental import pallas as pl
from jax.experimental.pallas import tpu as pltpu

BATCH = 64
HEIGHT = 32
WIDTH = 32
EMBED_DIM = 256
MAX_POS = 50

BLOCK_B = 2


def _body(row_ref, col_ref, out_ref):
    x = col_ref[:WIDTH, :]
    y = row_ref[:HEIGHT, :]
    left = jnp.broadcast_to(x[None, :, :], (HEIGHT, WIDTH, EMBED_DIM))
    left = left.reshape(HEIGHT * WIDTH, EMBED_DIM)
    right = jnp.broadcast_to(y[:, None, :], (HEIGHT, WIDTH, EMBED_DIM))
    right = right.reshape(HEIGHT * WIDTH, EMBED_DIM)
    tile = jnp.concatenate([left, right], axis=-1)
    out_ref[...] = jnp.broadcast_to(tile[None], (BLOCK_B, HEIGHT * WIDTH, 2 * EMBED_DIM))


def kernel(row_embeddings, column_embeddings):
    out = pl.pallas_call(
        _body,
        grid=(BATCH // BLOCK_B,),
        in_specs=[
            pl.BlockSpec((MAX_POS, EMBED_DIM), lambda b: (0, 0)),
            pl.BlockSpec((MAX_POS, EMBED_DIM), lambda b: (0, 0)),
        ],
        out_specs=pl.BlockSpec(
            (BLOCK_B, HEIGHT * WIDTH, 2 * EMBED_DIM), lambda b: (b, 0, 0)
        ),
        out_shape=jax.ShapeDtypeStruct(
            (BATCH, HEIGHT * WIDTH, 2 * EMBED_DIM), jnp.float32
        ),
        compiler_params=pltpu.CompilerParams(
            dimension_semantics=("arbitrary",),
        ),
    )(row_embeddings, column_embeddings)
    return out
